# baseline (speedup 1.0000x reference)
import functools

import jax
import jax.numpy as jnp
from jax import lax
from jax.experimental import pallas as pl
from jax.experimental.pallas import tpu as pltpu
from jax.experimental.pallas import tpu_sc as plsc

V = 100000
D = 64
DP = 128
B = 16384
K = 512

NC = 2
NS = 16
NW = NC * NS
BPW = B // NW
CH = 128
NCH = BPW // CH
KPC = K // NC
NKCH = KPC // CH
PAD = 17
L = 16
QN = D // L
TBLK = 1024
VP = 51200


def _sc_body(center, pos, neg, in_tab, out_tab,
             scores_hbm, sums_hbm,
             cidx, pidx, nidx, cidx2, pidx2, nidx2,
             in_buf, pos_buf, neg_buf,
             part, scores_v, sums_v,
             sem, sem2, nsem):
    cid = lax.axis_index("c")
    sid = lax.axis_index("s")
    wid = sid * NC + cid
    base = wid * BPW

    for j in range(NCH):
        pltpu.sync_copy(center.at[pl.ds(base + j * CH, CH)], cidx.at[j])
        pltpu.sync_copy(pos.at[pl.ds(base + j * CH, CH)], pidx.at[j])

    half = VP
    for j in range(NCH):
        for q in range(CH // L):
            cv = cidx[j, pl.ds(q * L, L)]
            pv = pidx[j, pl.ds(q * L, L)]
            cidx2[j, pl.ds(q * L, L)] = jnp.where(cv >= half, cv - half, cv)
            pidx2[j, pl.ds(q * L, L)] = jnp.where(pv >= half, pv - half, pv)

    def fire(j):
        bsel = j % 2
        s = sem if bsel == 0 else sem2
        a = pltpu.async_copy(in_tab.at[cidx2.at[j]], in_buf.at[bsel], s)
        b = pltpu.async_copy(out_tab.at[pidx2.at[j]], pos_buf.at[bsel], s)
        return (a, b)

    copies = {0: fire(0)}

    lanes = lax.iota(jnp.int32, L)
    zero = jnp.zeros((L,), jnp.float32)
    UNROLL = 4

    sacc = [zero] * QN
    for j in range(NCH):
        if j + 1 < NCH:
            copies[j + 1] = fire(j + 1)
        for c in copies[j]:
            c.wait()
        bsel = j % 2

        def prod_body(i, carry, _j=j, _bsel=bsel):
            sq = list(carry)
            cvec = jnp.where(cidx[_j, pl.ds(i * L, L)] >= VP, D, 0)
            pvec = jnp.where(pidx[_j, pl.ds(i * L, L)] >= VP, D, 0)
            for u in range(L):
                b = i * L + u
                coff = cvec[u]
                poff = pvec[u]
                iv0 = in_buf[_bsel, b, pl.ds(coff, L)]
                acc = iv0 * pos_buf[_bsel, b, pl.ds(poff, L)]
                sq[0] += iv0
                for q in range(1, QN):
                    ivq = in_buf[_bsel, b, pl.ds(coff + q * L, L)]
                    acc += ivq * pos_buf[_bsel, b, pl.ds(poff + q * L, L)]
                    sq[q] += ivq
                plsc.store_scatter(part, [(_j * CH + b) * PAD + lanes], acc)
            return tuple(sq)

        sacc = list(lax.fori_loop(0, CH // L, prod_body, tuple(sacc)))

    for q in range(QN):
        sums_v[0, pl.ds(q * L, L)] = sacc[q]
        sums_v[1, pl.ds(q * L, L)] = zero

    @pl.when(sid == 0)
    def _():
        for j in range(NKCH):
            pltpu.sync_copy(neg.at[pl.ds(cid * KPC + j * CH, CH)], nidx.at[j])
        for j in range(NKCH):
            for q in range(CH // L):
                nv = nidx[j, pl.ds(q * L, L)]
                nidx2[j, pl.ds(q * L, L)] = jnp.where(
                    nv >= VP, nv - VP, nv)
        nc0 = pltpu.async_copy(out_tab.at[nidx2.at[0]], neg_buf.at[0], nsem)
        nc1 = pltpu.async_copy(out_tab.at[nidx2.at[1]], neg_buf.at[1], nsem)
        nc0.wait()
        nc1.wait()

        def neg_body(i, carry):
            nq = list(carry)
            nv0 = jnp.where(nidx[0, pl.ds(i * L, L)] >= VP, D, 0)
            nv1 = jnp.where(nidx[1, pl.ds(i * L, L)] >= VP, D, 0)
            for u in range(L):
                b = i * L + u
                noff0 = nv0[u]
                noff1 = nv1[u]
                for q in range(QN):
                    nq[q] += (neg_buf[0, b, pl.ds(noff0 + q * L, L)] +
                              neg_buf[1, b, pl.ds(noff1 + q * L, L)])
            return tuple(nq)

        nacc = lax.fori_loop(0, CH // L, neg_body, (zero,) * QN)
        for q in range(QN):
            sums_v[1, pl.ds(q * L, L)] = nacc[q]

    def red_body(t, carry):
        rowbase = (t * L + lanes) * PAD
        acc = plsc.load_gather(part, [rowbase])
        for jcol in range(1, L):
            acc += plsc.load_gather(part, [rowbase + jcol])
        scores_v[pl.ds(t * L, L)] = acc
        return carry

    lax.fori_loop(0, BPW // L, red_body, 0)

    pltpu.sync_copy(scores_v, scores_hbm.at[pl.ds(base, BPW)])
    pltpu.sync_copy(sums_v, sums_hbm.at[wid])


@functools.partial(
    pl.kernel,
    out_type=(
        jax.ShapeDtypeStruct((B,), jnp.float32),
        jax.ShapeDtypeStruct((NW, 2, D), jnp.float32),
    ),
    mesh=plsc.VectorSubcoreMesh(core_axis_name="c", subcore_axis_name="s"),
    compiler_params=pltpu.CompilerParams(
        needs_layout_passes=False, use_tc_tiling_on_sc=False),
    scratch_types=[
        pltpu.VMEM((NCH, CH), jnp.int32),
        pltpu.VMEM((NCH, CH), jnp.int32),
        pltpu.VMEM((NKCH, CH), jnp.int32),
        pltpu.VMEM((NCH, CH), jnp.int32),
        pltpu.VMEM((NCH, CH), jnp.int32),
        pltpu.VMEM((NKCH, CH), jnp.int32),
        pltpu.VMEM((2, CH, DP), jnp.float32),
        pltpu.VMEM((2, CH, DP), jnp.float32),
        pltpu.VMEM((2, CH, DP), jnp.float32),
        pltpu.VMEM((BPW * PAD,), jnp.float32),
        pltpu.VMEM((BPW,), jnp.float32),
        pltpu.VMEM((2, D), jnp.float32),
        pltpu.SemaphoreType.DMA,
        pltpu.SemaphoreType.DMA,
        pltpu.SemaphoreType.DMA,
    ],
)
def _sc_stage(center, pos, neg, in_tab, out_tab, scores_hbm, sums_hbm,
              cidx, pidx, nidx, cidx2, pidx2, nidx2, in_buf, pos_buf, neg_buf,
              part, scores_v, sums_v, sem, sem2, nsem):
    _sc_body(center, pos, neg, in_tab, out_tab, scores_hbm, sums_hbm,
             cidx, pidx, nidx, cidx2, pidx2, nidx2, in_buf, pos_buf, neg_buf,
             part, scores_v, sums_v, sem, sem2, nsem)


def _tr_body(in_lo, in_hi, out_lo, out_hi, o_in_ref, o_out_ref):
    o_in_ref[:, 0:D] = in_lo[...].T
    o_in_ref[:, D:DP] = in_hi[...].T
    o_out_ref[:, 0:D] = out_lo[...].T
    o_out_ref[:, D:DP] = out_hi[...].T


def _transpose_pack(in_t, out_t):
    nblk = VP // TBLK
    last = (V - 1) // TBLK
    lo = pl.BlockSpec((D, TBLK), lambda i: (0, i))
    hi = pl.BlockSpec((D, TBLK), lambda i: (0, jnp.minimum(i + VP // TBLK, last)))
    return pl.pallas_call(
        _tr_body,
        grid=(nblk,),
        in_specs=[lo, hi, lo, hi],
        out_specs=[pl.BlockSpec((TBLK, DP), lambda i: (i, 0)),
                   pl.BlockSpec((TBLK, DP), lambda i: (i, 0))],
        out_shape=[jax.ShapeDtypeStruct((VP, DP), jnp.float32),
                   jax.ShapeDtypeStruct((VP, DP), jnp.float32)],
    )(in_t, in_t, out_t, out_t)


def _tc_body(scores_ref, sums_ref, out_ref):
    s = scores_ref[...]
    total = jnp.sum(jax.nn.log_sigmoid(s))
    sm = sums_ref[...]
    c = jnp.sum(sm, axis=0, keepdims=True)
    ns = jnp.sum(c[:, 0:D] * c[:, D:2 * D])
    out_ref[...] = jnp.reshape(-(total + B * jax.nn.log_sigmoid(-ns)), (1, 1))


def kernel(center_word, positive_words, negative_words, input_table, output_table):
    in_tab, out_tab = _transpose_pack(input_table.T, output_table.T)
    scores, sums = _sc_stage(
        center_word.astype(jnp.int32),
        positive_words.astype(jnp.int32),
        negative_words.astype(jnp.int32),
        in_tab, out_tab)
    out = pl.pallas_call(
        _tc_body,
        out_shape=jax.ShapeDtypeStruct((1, 1), jnp.float32),
    )(scores.reshape(128, 128), sums.reshape(NW, 2 * D))
    return out[0, 0]


# speedup vs baseline: 1.3257x; 1.3257x over previous
import functools

import jax
import jax.numpy as jnp
from jax import lax
from jax.experimental import pallas as pl
from jax.experimental.pallas import tpu as pltpu
from jax.experimental.pallas import tpu_sc as plsc

V = 100000
D = 64
DP = 128
B = 16384
K = 512

NC = 2
NS = 16
NW = NC * NS
BPW = B // NW
CH = 128
NCH = BPW // CH
KPC = K // NC
NKCH = KPC // CH
PAD = 17
L = 16
QN = D // L
TBLK = 2048
VP = 51200


def _sc_body(center, pos, neg, in_tab, out_tab,
             scores_hbm, sums_hbm,
             cidx, pidx, nidx, cidx2, pidx2, nidx2,
             in_buf, pos_buf, neg_buf,
             part, scores_v, sums_v,
             sem, sem2, nsem):
    cid = lax.axis_index("c")
    sid = lax.axis_index("s")
    wid = sid * NC + cid
    base = wid * BPW

    for j in range(NCH):
        pltpu.sync_copy(center.at[pl.ds(base + j * CH, CH)], cidx.at[j])
        pltpu.sync_copy(pos.at[pl.ds(base + j * CH, CH)], pidx.at[j])

    half = VP
    for j in range(NCH):
        for q in range(CH // L):
            cv = cidx[j, pl.ds(q * L, L)]
            pv = pidx[j, pl.ds(q * L, L)]
            cidx2[j, pl.ds(q * L, L)] = jnp.where(cv >= half, cv - half, cv)
            pidx2[j, pl.ds(q * L, L)] = jnp.where(pv >= half, pv - half, pv)

    def fire(j):
        bsel = j % 2
        s = sem if bsel == 0 else sem2
        a = pltpu.async_copy(in_tab.at[cidx2.at[j]], in_buf.at[bsel], s)
        b = pltpu.async_copy(out_tab.at[pidx2.at[j]], pos_buf.at[bsel], s)
        return (a, b)

    copies = {0: fire(0)}

    lanes = lax.iota(jnp.int32, L)
    zero = jnp.zeros((L,), jnp.float32)
    UNROLL = 4

    sacc = [zero] * QN
    for j in range(NCH):
        if j + 1 < NCH:
            copies[j + 1] = fire(j + 1)
        for c in copies[j]:
            c.wait()
        bsel = j % 2

        def prod_body(i, carry, _j=j, _bsel=bsel):
            sq = list(carry)
            cvec = jnp.where(cidx[_j, pl.ds(i * L, L)] >= VP, D, 0)
            pvec = jnp.where(pidx[_j, pl.ds(i * L, L)] >= VP, D, 0)
            for u in range(L):
                b = i * L + u
                coff = cvec[u]
                poff = pvec[u]
                iv0 = in_buf[_bsel, b, pl.ds(coff, L)]
                acc = iv0 * pos_buf[_bsel, b, pl.ds(poff, L)]
                sq[0] += iv0
                for q in range(1, QN):
                    ivq = in_buf[_bsel, b, pl.ds(coff + q * L, L)]
                    acc += ivq * pos_buf[_bsel, b, pl.ds(poff + q * L, L)]
                    sq[q] += ivq
                plsc.store_scatter(part, [(_j * CH + b) * PAD + lanes], acc)
            return tuple(sq)

        sacc = list(lax.fori_loop(0, CH // L, prod_body, tuple(sacc)))

    for q in range(QN):
        sums_v[0, pl.ds(q * L, L)] = sacc[q]
        sums_v[1, pl.ds(q * L, L)] = zero

    @pl.when(sid == 0)
    def _():
        for j in range(NKCH):
            pltpu.sync_copy(neg.at[pl.ds(cid * KPC + j * CH, CH)], nidx.at[j])
        for j in range(NKCH):
            for q in range(CH // L):
                nv = nidx[j, pl.ds(q * L, L)]
                nidx2[j, pl.ds(q * L, L)] = jnp.where(
                    nv >= VP, nv - VP, nv)
        nc0 = pltpu.async_copy(out_tab.at[nidx2.at[0]], neg_buf.at[0], nsem)
        nc1 = pltpu.async_copy(out_tab.at[nidx2.at[1]], neg_buf.at[1], nsem)
        nc0.wait()
        nc1.wait()

        def neg_body(i, carry):
            nq = list(carry)
            nv0 = jnp.where(nidx[0, pl.ds(i * L, L)] >= VP, D, 0)
            nv1 = jnp.where(nidx[1, pl.ds(i * L, L)] >= VP, D, 0)
            for u in range(L):
                b = i * L + u
                noff0 = nv0[u]
                noff1 = nv1[u]
                for q in range(QN):
                    nq[q] += (neg_buf[0, b, pl.ds(noff0 + q * L, L)] +
                              neg_buf[1, b, pl.ds(noff1 + q * L, L)])
            return tuple(nq)

        nacc = lax.fori_loop(0, CH // L, neg_body, (zero,) * QN)
        for q in range(QN):
            sums_v[1, pl.ds(q * L, L)] = nacc[q]

    def red_body(t, carry):
        rowbase = (t * L + lanes) * PAD
        acc = plsc.load_gather(part, [rowbase])
        for jcol in range(1, L):
            acc += plsc.load_gather(part, [rowbase + jcol])
        scores_v[pl.ds(t * L, L)] = acc
        return carry

    lax.fori_loop(0, BPW // L, red_body, 0)

    pltpu.sync_copy(scores_v, scores_hbm.at[pl.ds(base, BPW)])
    pltpu.sync_copy(sums_v, sums_hbm.at[wid])


@functools.partial(
    pl.kernel,
    out_type=(
        jax.ShapeDtypeStruct((B,), jnp.float32),
        jax.ShapeDtypeStruct((NW, 2, D), jnp.float32),
    ),
    mesh=plsc.VectorSubcoreMesh(core_axis_name="c", subcore_axis_name="s"),
    compiler_params=pltpu.CompilerParams(
        needs_layout_passes=False, use_tc_tiling_on_sc=False),
    scratch_types=[
        pltpu.VMEM((NCH, CH), jnp.int32),
        pltpu.VMEM((NCH, CH), jnp.int32),
        pltpu.VMEM((NKCH, CH), jnp.int32),
        pltpu.VMEM((NCH, CH), jnp.int32),
        pltpu.VMEM((NCH, CH), jnp.int32),
        pltpu.VMEM((NKCH, CH), jnp.int32),
        pltpu.VMEM((2, CH, DP), jnp.float32),
        pltpu.VMEM((2, CH, DP), jnp.float32),
        pltpu.VMEM((2, CH, DP), jnp.float32),
        pltpu.VMEM((BPW * PAD,), jnp.float32),
        pltpu.VMEM((BPW,), jnp.float32),
        pltpu.VMEM((2, D), jnp.float32),
        pltpu.SemaphoreType.DMA,
        pltpu.SemaphoreType.DMA,
        pltpu.SemaphoreType.DMA,
    ],
)
def _sc_stage(center, pos, neg, in_tab, out_tab, scores_hbm, sums_hbm,
              cidx, pidx, nidx, cidx2, pidx2, nidx2, in_buf, pos_buf, neg_buf,
              part, scores_v, sums_v, sem, sem2, nsem):
    _sc_body(center, pos, neg, in_tab, out_tab, scores_hbm, sums_hbm,
             cidx, pidx, nidx, cidx2, pidx2, nidx2, in_buf, pos_buf, neg_buf,
             part, scores_v, sums_v, sem, sem2, nsem)


def _tr_body(in_lo, in_hi, out_lo, out_hi, o_in_ref, o_out_ref):
    o_in_ref[...] = jnp.concatenate([in_lo[...], in_hi[...]], axis=0).T
    o_out_ref[...] = jnp.concatenate([out_lo[...], out_hi[...]], axis=0).T


def _transpose_pack(in_t, out_t):
    nblk = VP // TBLK
    last = (V - 1) // TBLK
    lo = pl.BlockSpec((D, TBLK), lambda i: (0, i))
    hi = pl.BlockSpec((D, TBLK), lambda i: (0, jnp.minimum(i + VP // TBLK, last)))
    return pl.pallas_call(
        _tr_body,
        grid=(nblk,),
        in_specs=[lo, hi, lo, hi],
        out_specs=[pl.BlockSpec((TBLK, DP), lambda i: (i, 0)),
                   pl.BlockSpec((TBLK, DP), lambda i: (i, 0))],
        out_shape=[jax.ShapeDtypeStruct((VP, DP), jnp.float32),
                   jax.ShapeDtypeStruct((VP, DP), jnp.float32)],
    )(in_t, in_t, out_t, out_t)


def _tc_body(scores_ref, sums_ref, out_ref):
    s = scores_ref[...]
    total = jnp.sum(jax.nn.log_sigmoid(s))
    sm = sums_ref[...]
    c = jnp.sum(sm, axis=0, keepdims=True)
    ns = jnp.sum(c[:, 0:D] * c[:, D:2 * D])
    out_ref[...] = jnp.reshape(-(total + B * jax.nn.log_sigmoid(-ns)), (1, 1))


def kernel(center_word, positive_words, negative_words, input_table, output_table):
    in_tab, out_tab = _transpose_pack(input_table.T, output_table.T)
    scores, sums = _sc_stage(
        center_word.astype(jnp.int32),
        positive_words.astype(jnp.int32),
        negative_words.astype(jnp.int32),
        in_tab, out_tab)
    out = pl.pallas_call(
        _tc_body,
        out_shape=jax.ShapeDtypeStruct((1, 1), jnp.float32),
    )(scores.reshape(128, 128), sums.reshape(NW, 2 * D))
    return out[0, 0]


# speedup vs baseline: 1.3836x; 1.0436x over previous
import functools

import jax
import jax.numpy as jnp
from jax import lax
from jax.experimental import pallas as pl
from jax.experimental.pallas import tpu as pltpu
from jax.experimental.pallas import tpu_sc as plsc

V = 100000
D = 64
DP = 128
B = 16384
K = 512

NC = 2
NS = 16
NW = NC * NS
BPW = B // NW
CH = 128
NCH = BPW // CH
KPC = K // NC
NKCH = KPC // CH
PAD = 17
L = 16
QN = D // L
TBLK = 2048
VP = 51200


def _sc_body(center, pos, neg, in_tab, out_tab,
             scores_hbm, sums_hbm,
             cidx, pidx, nidx, cidx2, pidx2, nidx2,
             in_buf, pos_buf, neg_buf,
             part, scores_v, sums_v,
             sem, sem2, nsem):
    cid = lax.axis_index("c")
    sid = lax.axis_index("s")
    wid = sid * NC + cid
    base = wid * BPW

    pltpu.sync_copy(center.at[pl.ds(base, BPW)], cidx)
    pltpu.sync_copy(pos.at[pl.ds(base, BPW)], pidx)

    half = VP
    for k in range(BPW // L):
        cv = cidx[pl.ds(k * L, L)]
        pv = pidx[pl.ds(k * L, L)]
        cidx2[pl.ds(k * L, L)] = jnp.where(cv >= half, cv - half, cv)
        pidx2[pl.ds(k * L, L)] = jnp.where(pv >= half, pv - half, pv)

    def fire(j):
        bsel = j % 2
        s = sem if bsel == 0 else sem2
        a = pltpu.async_copy(
            in_tab.at[cidx2.at[pl.ds(j * CH, CH)]], in_buf.at[bsel], s)
        b = pltpu.async_copy(
            out_tab.at[pidx2.at[pl.ds(j * CH, CH)]], pos_buf.at[bsel], s)
        return (a, b)

    copies = {0: fire(0)}

    lanes = lax.iota(jnp.int32, L)
    zero = jnp.zeros((L,), jnp.float32)
    UNROLL = 4

    sacc = [zero] * QN
    for j in range(NCH):
        if j + 1 < NCH:
            copies[j + 1] = fire(j + 1)
        for c in copies[j]:
            c.wait()
        bsel = j % 2

        def prod_body(i, carry, _j=j, _bsel=bsel):
            sq = list(carry)
            cvec = jnp.where(cidx[pl.ds(_j * CH + i * L, L)] >= VP, D, 0)
            pvec = jnp.where(pidx[pl.ds(_j * CH + i * L, L)] >= VP, D, 0)
            for u in range(L):
                b = i * L + u
                coff = cvec[u]
                poff = pvec[u]
                iv0 = in_buf[_bsel, b, pl.ds(coff, L)]
                acc = iv0 * pos_buf[_bsel, b, pl.ds(poff, L)]
                sq[0] += iv0
                for q in range(1, QN):
                    ivq = in_buf[_bsel, b, pl.ds(coff + q * L, L)]
                    acc += ivq * pos_buf[_bsel, b, pl.ds(poff + q * L, L)]
                    sq[q] += ivq
                plsc.store_scatter(part, [(_j * CH + b) * PAD + lanes], acc)
            return tuple(sq)

        sacc = list(lax.fori_loop(0, CH // L, prod_body, tuple(sacc)))

    for q in range(QN):
        sums_v[0, pl.ds(q * L, L)] = sacc[q]
        sums_v[1, pl.ds(q * L, L)] = zero

    @pl.when(sid == 0)
    def _():
        pltpu.sync_copy(neg.at[pl.ds(cid * KPC, KPC)], nidx)
        for k in range(KPC // L):
            nv = nidx[pl.ds(k * L, L)]
            nidx2[pl.ds(k * L, L)] = jnp.where(nv >= VP, nv - VP, nv)
        nc0 = pltpu.async_copy(
            out_tab.at[nidx2.at[pl.ds(0, CH)]], neg_buf.at[0], nsem)
        nc1 = pltpu.async_copy(
            out_tab.at[nidx2.at[pl.ds(CH, CH)]], neg_buf.at[1], nsem)
        nc0.wait()
        nc1.wait()

        def neg_body(i, carry):
            nq = list(carry)
            nv0 = jnp.where(nidx[pl.ds(i * L, L)] >= VP, D, 0)
            nv1 = jnp.where(nidx[pl.ds(CH + i * L, L)] >= VP, D, 0)
            for u in range(L):
                b = i * L + u
                noff0 = nv0[u]
                noff1 = nv1[u]
                for q in range(QN):
                    nq[q] += (neg_buf[0, b, pl.ds(noff0 + q * L, L)] +
                              neg_buf[1, b, pl.ds(noff1 + q * L, L)])
            return tuple(nq)

        nacc = lax.fori_loop(0, CH // L, neg_body, (zero,) * QN)
        for q in range(QN):
            sums_v[1, pl.ds(q * L, L)] = nacc[q]

    def red_body(t, carry):
        rowbase = (t * L + lanes) * PAD
        acc = plsc.load_gather(part, [rowbase])
        for jcol in range(1, L):
            acc += plsc.load_gather(part, [rowbase + jcol])
        scores_v[pl.ds(t * L, L)] = acc
        return carry

    lax.fori_loop(0, BPW // L, red_body, 0)

    pltpu.sync_copy(scores_v, scores_hbm.at[pl.ds(base, BPW)])
    pltpu.sync_copy(sums_v, sums_hbm.at[wid])


@functools.partial(
    pl.kernel,
    out_type=(
        jax.ShapeDtypeStruct((B,), jnp.float32),
        jax.ShapeDtypeStruct((NW, 2, D), jnp.float32),
    ),
    mesh=plsc.VectorSubcoreMesh(core_axis_name="c", subcore_axis_name="s"),
    compiler_params=pltpu.CompilerParams(
        needs_layout_passes=False, use_tc_tiling_on_sc=False),
    scratch_types=[
        pltpu.VMEM((BPW,), jnp.int32),
        pltpu.VMEM((BPW,), jnp.int32),
        pltpu.VMEM((KPC,), jnp.int32),
        pltpu.VMEM((BPW,), jnp.int32),
        pltpu.VMEM((BPW,), jnp.int32),
        pltpu.VMEM((KPC,), jnp.int32),
        pltpu.VMEM((2, CH, DP), jnp.float32),
        pltpu.VMEM((2, CH, DP), jnp.float32),
        pltpu.VMEM((2, CH, DP), jnp.float32),
        pltpu.VMEM((BPW * PAD,), jnp.float32),
        pltpu.VMEM((BPW,), jnp.float32),
        pltpu.VMEM((2, D), jnp.float32),
        pltpu.SemaphoreType.DMA,
        pltpu.SemaphoreType.DMA,
        pltpu.SemaphoreType.DMA,
    ],
)
def _sc_stage(center, pos, neg, in_tab, out_tab, scores_hbm, sums_hbm,
              cidx, pidx, nidx, cidx2, pidx2, nidx2, in_buf, pos_buf, neg_buf,
              part, scores_v, sums_v, sem, sem2, nsem):
    _sc_body(center, pos, neg, in_tab, out_tab, scores_hbm, sums_hbm,
             cidx, pidx, nidx, cidx2, pidx2, nidx2, in_buf, pos_buf, neg_buf,
             part, scores_v, sums_v, sem, sem2, nsem)


def _tr_body(in_lo, in_hi, out_lo, out_hi, o_in_ref, o_out_ref):
    o_in_ref[...] = jnp.concatenate([in_lo[...], in_hi[...]], axis=0).T
    o_out_ref[...] = jnp.concatenate([out_lo[...], out_hi[...]], axis=0).T


def _transpose_pack(in_t, out_t):
    nblk = VP // TBLK
    last = (V - 1) // TBLK
    lo = pl.BlockSpec((D, TBLK), lambda i: (0, i))
    hi = pl.BlockSpec((D, TBLK), lambda i: (0, jnp.minimum(i + VP // TBLK, last)))
    return pl.pallas_call(
        _tr_body,
        grid=(nblk,),
        in_specs=[lo, hi, lo, hi],
        out_specs=[pl.BlockSpec((TBLK, DP), lambda i: (i, 0)),
                   pl.BlockSpec((TBLK, DP), lambda i: (i, 0))],
        out_shape=[jax.ShapeDtypeStruct((VP, DP), jnp.float32),
                   jax.ShapeDtypeStruct((VP, DP), jnp.float32)],
    )(in_t, in_t, out_t, out_t)


def _tc_body(scores_ref, sums_ref, out_ref):
    s = scores_ref[...]
    total = jnp.sum(jax.nn.log_sigmoid(s))
    sm = sums_ref[...]
    c = jnp.sum(sm, axis=0, keepdims=True)
    ns = jnp.sum(c[:, 0:D] * c[:, D:2 * D])
    out_ref[...] = jnp.reshape(-(total + B * jax.nn.log_sigmoid(-ns)), (1, 1))


def kernel(center_word, positive_words, negative_words, input_table, output_table):
    in_tab, out_tab = _transpose_pack(input_table.T, output_table.T)
    scores, sums = _sc_stage(
        center_word.astype(jnp.int32),
        positive_words.astype(jnp.int32),
        negative_words.astype(jnp.int32),
        in_tab, out_tab)
    out = pl.pallas_call(
        _tc_body,
        out_shape=jax.ShapeDtypeStruct((1, 1), jnp.float32),
    )(scores.reshape(128, 128), sums.reshape(NW, 2 * D))
    return out[0, 0]
